# baseline (speedup 1.0000x reference)
import functools

import jax
import jax.numpy as jnp
from jax.experimental import pallas as pl
from jax.experimental.pallas import tpu as pltpu

_N = 10000
_E = 160000
_D = 32

_BN = 2000
_BE = 2000


def _prologue_body(x_ref, w0t_ref, b0_ref, o_ref):
    o_ref[...] = jnp.maximum(
        jnp.dot(x_ref[...], w0t_ref[...], preferred_element_type=jnp.float32)
        + b0_ref[...], 0.0)


def _hidden_body(ea_ref, wn1t_ref, bn1_ref, o_ref):
    o_ref[...] = jnp.maximum(
        jnp.dot(ea_ref[...], wn1t_ref[...], preferred_element_type=jnp.float32)
        + bn1_ref[...], 0.0)


def _msg_body(osrc_ref, hid_ref, wn2t_ref, bn2m_ref, msg_ref):
    osrc = osrc_ref[...]
    wf = jnp.dot(hid_ref[...], wn2t_ref[...],
                 preferred_element_type=jnp.float32)
    acc = jnp.dot(osrc, bn2m_ref[...],
                  preferred_element_type=jnp.float32)
    for i in range(_D):
        acc = acc + osrc[:, i:i + 1] * wf[:, _D * i:_D * (i + 1)]
    msg_ref[...] = acc


def _update_body(agg_ref, rdeg_ref, h_ref, root_ref, bias_ref,
                 wiht_ref, bih_ref, whht_ref, bhh_ref, h2_ref):
    h = h_ref[...]
    agg = agg_ref[...] * rdeg_ref[...]
    m = jnp.maximum(
        agg + jnp.dot(h, root_ref[...], preferred_element_type=jnp.float32)
        + bias_ref[...], 0.0)
    gi = jnp.dot(m, wiht_ref[...], preferred_element_type=jnp.float32) + bih_ref[...]
    gh = jnp.dot(h, whht_ref[...], preferred_element_type=jnp.float32) + bhh_ref[...]
    r = jax.nn.sigmoid(gi[:, :_D] + gh[:, :_D])
    z = jax.nn.sigmoid(gi[:, _D:2 * _D] + gh[:, _D:2 * _D])
    n = jnp.tanh(gi[:, 2 * _D:] + r * gh[:, 2 * _D:])
    h2_ref[...] = (1.0 - z) * n + z * h


def _epilogue_body(h_ref, w1t_ref, b1_ref, w2_ref, b2_ref, y_ref):
    t = jnp.maximum(
        jnp.dot(h_ref[...], w1t_ref[...], preferred_element_type=jnp.float32)
        + b1_ref[...], 0.0)
    y_ref[...] = jnp.sum(t * w2_ref[...], axis=1, keepdims=True) + b2_ref[...]


def _full(shape):
    return pl.BlockSpec(shape, lambda i: (0,) * len(shape))


def kernel(x, edge_index, edge_attr, W0, b0, Wn1, bn1, Wn2, bn2, root, bias,
           Wih, Whh, bih, bhh, W1, b1, W2, b2):
    src = edge_index[0]
    dst = edge_index[1]

    out = pl.pallas_call(
        _prologue_body,
        grid=(_N // _BN,),
        in_specs=[pl.BlockSpec((_BN, 128), lambda i: (i, 0)),
                  _full((128, _D)), _full((1, _D))],
        out_specs=pl.BlockSpec((_BN, _D), lambda i: (i, 0)),
        out_shape=jax.ShapeDtypeStruct((_N, _D), jnp.float32),
    )(x, W0.T, b0.reshape(1, _D))

    hidden = pl.pallas_call(
        _hidden_body,
        grid=(_E // _BE,),
        in_specs=[pl.BlockSpec((_BE, 16), lambda i: (i, 0)),
                  _full((16, _D)), _full((1, _D))],
        out_specs=pl.BlockSpec((_BE, _D), lambda i: (i, 0)),
        out_shape=jax.ShapeDtypeStruct((_E, _D), jnp.float32),
    )(edge_attr, Wn1.T, bn1.reshape(1, _D))

    deg = jax.ops.segment_sum(jnp.ones((_E,), jnp.float32), dst, num_segments=_N)
    rdeg = (1.0 / jnp.clip(deg, 1.0)).reshape(_N, 1)

    wn2t = Wn2.T
    bn2m = bn2.reshape(_D, _D)

    msg_call = pl.pallas_call(
        _msg_body,
        grid=(_E // _BE,),
        in_specs=[pl.BlockSpec((_BE, _D), lambda i: (i, 0)),
                  pl.BlockSpec((_BE, _D), lambda i: (i, 0)),
                  _full((_D, _D * _D)), _full((_D, _D))],
        out_specs=pl.BlockSpec((_BE, _D), lambda i: (i, 0)),
        out_shape=jax.ShapeDtypeStruct((_E, _D), jnp.float32),
    )

    upd_call = pl.pallas_call(
        _update_body,
        grid=(_N // _BN,),
        in_specs=[pl.BlockSpec((_BN, _D), lambda i: (i, 0)),
                  pl.BlockSpec((_BN, 1), lambda i: (i, 0)),
                  pl.BlockSpec((_BN, _D), lambda i: (i, 0)),
                  _full((_D, _D)), _full((1, _D)),
                  _full((_D, 3 * _D)), _full((1, 3 * _D)),
                  _full((_D, 3 * _D)), _full((1, 3 * _D))],
        out_specs=pl.BlockSpec((_BN, _D), lambda i: (i, 0)),
        out_shape=jax.ShapeDtypeStruct((_N, _D), jnp.float32),
    )

    rootm = root
    biasr = bias.reshape(1, _D)
    wiht = Wih.T
    bihr = bih.reshape(1, 3 * _D)
    whht = Whh.T
    bhhr = bhh.reshape(1, 3 * _D)

    for _ in range(3):
        out_src = jnp.take(out, src, axis=0)
        msg = msg_call(out_src, hidden, wn2t, bn2m)
        aggs = jax.ops.segment_sum(msg, dst, num_segments=_N)
        out = upd_call(aggs, rdeg, out, rootm, biasr,
                       wiht, bihr, whht, bhhr)

    y = pl.pallas_call(
        _epilogue_body,
        grid=(_N // _BN,),
        in_specs=[pl.BlockSpec((_BN, _D), lambda i: (i, 0)),
                  _full((_D, _D)), _full((1, _D)),
                  _full((1, _D)), _full((1, 1))],
        out_specs=pl.BlockSpec((_BN, 1), lambda i: (i, 0)),
        out_shape=jax.ShapeDtypeStruct((_N, 1), jnp.float32),
    )(out, W1.T, b1.reshape(1, _D), W2, b2.reshape(1, 1))
    return y


# speedup vs baseline: 1.3573x; 1.3573x over previous
import functools

import jax
import jax.numpy as jnp
from jax import lax
from jax.experimental import pallas as pl
from jax.experimental.pallas import tpu as pltpu
from jax.experimental.pallas import tpu_sc as plsc

_N = 10000
_E = 160000
_D = 32

_BN = 2000
_BE = 2048

_NW = 32
_CHUNK = 128
_NCHUNK = 40
_EPW = _CHUNK * _NCHUNK
_EPAD = _NW * _EPW
_NPAD = 10240
_STRIPE = _NPAD // 16


def _prologue_body(x_ref, w0t_ref, b0_ref, o_ref):
    o_ref[...] = jnp.maximum(
        jnp.dot(x_ref[...], w0t_ref[...], preferred_element_type=jnp.float32)
        + b0_ref[...], 0.0)


def _hidden_body(ea_ref, wn1t_ref, bn1_ref, o_ref):
    o_ref[...] = jnp.maximum(
        jnp.dot(ea_ref[...], wn1t_ref[...], preferred_element_type=jnp.float32)
        + bn1_ref[...], 0.0)


def _msg_body(osrc_ref, hid_ref, wn2t_ref, bn2m_ref, msg_ref):
    osrc = osrc_ref[...]
    wf = jnp.dot(hid_ref[...], wn2t_ref[...],
                 preferred_element_type=jnp.float32)
    acc = jnp.dot(osrc, bn2m_ref[...],
                  preferred_element_type=jnp.float32)
    for i in range(_D):
        acc = acc + osrc[:, i:i + 1] * wf[:, _D * i:_D * (i + 1)]
    msg_ref[...] = acc


def _gather_body(tab_hbm, idx_hbm, out_hbm, idx_v, rows_v, sem):
    wid = lax.axis_index("s") * 2 + lax.axis_index("c")
    base = wid * _EPW

    def body(j, carry):
        start = base + j * _CHUNK
        pltpu.sync_copy(idx_hbm.at[pl.ds(start, _CHUNK)], idx_v)
        pltpu.async_copy(tab_hbm.at[idx_v], rows_v, sem).wait()
        pltpu.sync_copy(rows_v, out_hbm.at[pl.ds(start, _CHUNK)])
        return carry

    lax.fori_loop(0, _NCHUNK, body, 0)


def _scatter_body(msg_hbm, idx3_hbm, zeros_hbm, out_hbm, idx_v, rows_v, acc):
    cid = lax.axis_index("c")
    sid = lax.axis_index("s")
    wid = sid * 2 + cid
    pltpu.sync_copy(zeros_hbm.at[pl.ds(sid * _STRIPE, _STRIPE)],
                    acc.at[pl.ds(sid * _STRIPE, _STRIPE)])
    plsc.subcore_barrier()
    pltpu.sync_copy(idx3_hbm.at[wid], idx_v)

    def body(j, carry):
        pltpu.sync_copy(msg_hbm.at[pl.ds(wid * _EPW + j * _CHUNK, _CHUNK)],
                        rows_v)
        pltpu.sync_copy(rows_v, acc.at[idx_v.at[j]], add=True)
        return carry

    lax.fori_loop(0, _NCHUNK, body, 0)
    plsc.subcore_barrier()
    pltpu.sync_copy(acc.at[pl.ds(sid * _STRIPE, _STRIPE)],
                    out_hbm.at[cid, pl.ds(sid * _STRIPE, _STRIPE)])


_sc_mesh = plsc.VectorSubcoreMesh(core_axis_name="c", subcore_axis_name="s")

_gather = pl.kernel(
    _gather_body, mesh=_sc_mesh,
    out_type=jax.ShapeDtypeStruct((_EPAD, _D), jnp.float32),
    scratch_types=[pltpu.VMEM((_CHUNK,), jnp.int32),
                   pltpu.VMEM((_CHUNK, _D), jnp.float32),
                   pltpu.SemaphoreType.DMA],
    compiler_params=pltpu.CompilerParams(use_tc_tiling_on_sc=False),
)

_scatter = pl.kernel(
    _scatter_body, mesh=_sc_mesh,
    out_type=jax.ShapeDtypeStruct((2, _NPAD, _D), jnp.float32),
    scratch_types=[pltpu.VMEM((_NCHUNK, _CHUNK), jnp.int32),
                   pltpu.VMEM((_CHUNK, _D), jnp.float32),
                   pltpu.VMEM_SHARED((_NPAD, _D), jnp.float32)],
    compiler_params=pltpu.CompilerParams(use_tc_tiling_on_sc=False),
)


def _update_body(agg0_ref, agg1_ref, deg0_ref, deg1_ref, h_ref, root_ref,
                 bias_ref, wiht_ref, bih_ref, whht_ref, bhh_ref, h2_ref):
    h = h_ref[...]
    deg = jnp.maximum(deg0_ref[...] + deg1_ref[...], 1.0)
    agg = (agg0_ref[...] + agg1_ref[...]) / deg
    m = jnp.maximum(
        agg + jnp.dot(h, root_ref[...], preferred_element_type=jnp.float32)
        + bias_ref[...], 0.0)
    gi = jnp.dot(m, wiht_ref[...], preferred_element_type=jnp.float32) + bih_ref[...]
    gh = jnp.dot(h, whht_ref[...], preferred_element_type=jnp.float32) + bhh_ref[...]
    r = jax.nn.sigmoid(gi[:, :_D] + gh[:, :_D])
    z = jax.nn.sigmoid(gi[:, _D:2 * _D] + gh[:, _D:2 * _D])
    n = jnp.tanh(gi[:, 2 * _D:] + r * gh[:, 2 * _D:])
    h2_ref[...] = (1.0 - z) * n + z * h


def _epilogue_body(h_ref, w1t_ref, b1_ref, w2_ref, b2_ref, y_ref):
    t = jnp.maximum(
        jnp.dot(h_ref[...], w1t_ref[...], preferred_element_type=jnp.float32)
        + b1_ref[...], 0.0)
    y_ref[...] = jnp.sum(t * w2_ref[...], axis=1, keepdims=True) + b2_ref[...]


def _full(shape):
    return pl.BlockSpec(shape, lambda i: (0,) * len(shape))


def kernel(x, edge_index, edge_attr, W0, b0, Wn1, bn1, Wn2, bn2, root, bias,
           Wih, Whh, bih, bhh, W1, b1, W2, b2):
    pad = _EPAD - _E
    src = jnp.concatenate([edge_index[0], jnp.zeros((pad,), jnp.int32)])
    dst = jnp.concatenate([edge_index[1],
                           jnp.full((pad,), _N, jnp.int32)])
    dst3 = dst.reshape(_NW, _NCHUNK, _CHUNK)
    ea_pad = jnp.concatenate(
        [edge_attr, jnp.zeros((pad, edge_attr.shape[1]), jnp.float32)])
    zeros_acc = jnp.zeros((_NPAD, _D), jnp.float32)
    ones_e = jnp.ones((_EPAD, _D), jnp.float32)

    out = pl.pallas_call(
        _prologue_body,
        grid=(_N // _BN,),
        in_specs=[pl.BlockSpec((_BN, 128), lambda i: (i, 0)),
                  _full((128, _D)), _full((1, _D))],
        out_specs=pl.BlockSpec((_BN, _D), lambda i: (i, 0)),
        out_shape=jax.ShapeDtypeStruct((_N, _D), jnp.float32),
    )(x, W0.T, b0.reshape(1, _D))

    hidden = pl.pallas_call(
        _hidden_body,
        grid=(_EPAD // _BE,),
        in_specs=[pl.BlockSpec((_BE, 16), lambda i: (i, 0)),
                  _full((16, _D)), _full((1, _D))],
        out_specs=pl.BlockSpec((_BE, _D), lambda i: (i, 0)),
        out_shape=jax.ShapeDtypeStruct((_EPAD, _D), jnp.float32),
    )(ea_pad, Wn1.T, bn1.reshape(1, _D))

    degp = _scatter(ones_e, dst3, zeros_acc)
    deg0 = degp[0, :_N]
    deg1 = degp[1, :_N]

    wn2t = Wn2.T
    bn2m = bn2.reshape(_D, _D)

    msg_call = pl.pallas_call(
        _msg_body,
        grid=(_EPAD // _BE,),
        in_specs=[pl.BlockSpec((_BE, _D), lambda i: (i, 0)),
                  pl.BlockSpec((_BE, _D), lambda i: (i, 0)),
                  _full((_D, _D * _D)), _full((_D, _D))],
        out_specs=pl.BlockSpec((_BE, _D), lambda i: (i, 0)),
        out_shape=jax.ShapeDtypeStruct((_EPAD, _D), jnp.float32),
    )

    upd_call = pl.pallas_call(
        _update_body,
        grid=(_N // _BN,),
        in_specs=[pl.BlockSpec((_BN, _D), lambda i: (i, 0)),
                  pl.BlockSpec((_BN, _D), lambda i: (i, 0)),
                  pl.BlockSpec((_BN, _D), lambda i: (i, 0)),
                  pl.BlockSpec((_BN, _D), lambda i: (i, 0)),
                  pl.BlockSpec((_BN, _D), lambda i: (i, 0)),
                  _full((_D, _D)), _full((1, _D)),
                  _full((_D, 3 * _D)), _full((1, 3 * _D)),
                  _full((_D, 3 * _D)), _full((1, 3 * _D))],
        out_specs=pl.BlockSpec((_BN, _D), lambda i: (i, 0)),
        out_shape=jax.ShapeDtypeStruct((_N, _D), jnp.float32),
    )

    rootm = root
    biasr = bias.reshape(1, _D)
    wiht = Wih.T
    bihr = bih.reshape(1, 3 * _D)
    whht = Whh.T
    bhhr = bhh.reshape(1, 3 * _D)

    for _ in range(3):
        out_src = _gather(out, src)
        msg = msg_call(out_src, hidden, wn2t, bn2m)
        aggp = _scatter(msg, dst3, zeros_acc)
        out = upd_call(aggp[0, :_N], aggp[1, :_N], deg0, deg1, out,
                       rootm, biasr, wiht, bihr, whht, bhhr)

    y = pl.pallas_call(
        _epilogue_body,
        grid=(_N // _BN,),
        in_specs=[pl.BlockSpec((_BN, _D), lambda i: (i, 0)),
                  _full((_D, _D)), _full((1, _D)),
                  _full((1, _D)), _full((1, 1))],
        out_specs=pl.BlockSpec((_BN, 1), lambda i: (i, 0)),
        out_shape=jax.ShapeDtypeStruct((_N, 1), jnp.float32),
    )(out, W1.T, b1.reshape(1, _D), W2, b2.reshape(1, 1))
    return y


# speedup vs baseline: 1.3904x; 1.0244x over previous
import functools

import jax
import jax.numpy as jnp
from jax import lax
from jax.experimental import pallas as pl
from jax.experimental.pallas import tpu as pltpu
from jax.experimental.pallas import tpu_sc as plsc

_N = 10000
_E = 160000
_D = 32

_BN = 2000
_BE = 2048

_NW = 32
_CHUNK = 128
_NCHUNK = 40
_EPW = _CHUNK * _NCHUNK
_EPAD = _NW * _EPW
_NPAD = 10240
_STRIPE = _NPAD // 16


def _prologue_body(x_ref, w0t_ref, b0_ref, o_ref):
    o_ref[...] = jnp.maximum(
        jnp.dot(x_ref[...], w0t_ref[...], preferred_element_type=jnp.float32)
        + b0_ref[...], 0.0)


def _hidden_body(ea_ref, wn1t_ref, bn1_ref, o_ref):
    o_ref[...] = jnp.maximum(
        jnp.dot(ea_ref[...], wn1t_ref[...], preferred_element_type=jnp.float32)
        + bn1_ref[...], 0.0)


def _msg_body(osrc_ref, hid_ref, wn2t_ref, bn2m_ref, msg_ref):
    osrc = osrc_ref[...]
    wf = jnp.dot(hid_ref[...], wn2t_ref[...],
                 preferred_element_type=jnp.float32)
    acc = jnp.dot(osrc, bn2m_ref[...],
                  preferred_element_type=jnp.float32)
    for i in range(_D):
        acc = acc + osrc[:, i:i + 1] * wf[:, _D * i:_D * (i + 1)]
    msg_ref[...] = acc


_NB = 8
_NROUND = _NCHUNK // _NB


def _gather_body(tab_hbm, idx_hbm, out_hbm, idx_v, *rest):
    rows = rest[:_NB]
    semg, sems = rest[_NB], rest[_NB + 1]
    wid = lax.axis_index("s") * 2 + lax.axis_index("c")
    base = wid * _EPW
    pltpu.sync_copy(idx_hbm.at[wid], idx_v)

    def round_body(r, carry):
        cps = [pltpu.async_copy(tab_hbm.at[idx_v.at[r * _NB + b]],
                                rows[b], semg) for b in range(_NB)]
        for c in cps:
            c.wait()
        outs = [pltpu.async_copy(
            rows[b],
            out_hbm.at[pl.ds(base + (r * _NB + b) * _CHUNK, _CHUNK)],
            sems) for b in range(_NB)]
        for c in outs:
            c.wait()
        return carry

    lax.fori_loop(0, _NROUND, round_body, 0)


def _scatter_body(msg_hbm, idx3_hbm, zeros_hbm, out_hbm, idx_v, *rest):
    rows = rest[:_NB]
    semg, acc = rest[_NB], rest[_NB + 1]
    cid = lax.axis_index("c")
    sid = lax.axis_index("s")
    wid = sid * 2 + cid
    pltpu.sync_copy(zeros_hbm.at[pl.ds(sid * _STRIPE, _STRIPE)],
                    acc.at[pl.ds(sid * _STRIPE, _STRIPE)])
    plsc.subcore_barrier()
    pltpu.sync_copy(idx3_hbm.at[wid], idx_v)

    def round_body(r, carry):
        cps = [pltpu.async_copy(
            msg_hbm.at[pl.ds(wid * _EPW + (r * _NB + b) * _CHUNK, _CHUNK)],
            rows[b], semg) for b in range(_NB)]
        for c in cps:
            c.wait()
        for b in range(_NB):
            pltpu.sync_copy(rows[b], acc.at[idx_v.at[r * _NB + b]], add=True)
        return carry

    lax.fori_loop(0, _NROUND, round_body, 0)
    plsc.subcore_barrier()
    pltpu.sync_copy(acc.at[pl.ds(sid * _STRIPE, _STRIPE)],
                    out_hbm.at[cid, pl.ds(sid * _STRIPE, _STRIPE)])


_sc_mesh = plsc.VectorSubcoreMesh(core_axis_name="c", subcore_axis_name="s")

_gather = pl.kernel(
    _gather_body, mesh=_sc_mesh,
    out_type=jax.ShapeDtypeStruct((_EPAD, _D), jnp.float32),
    scratch_types=([pltpu.VMEM((_NCHUNK, _CHUNK), jnp.int32)]
                   + [pltpu.VMEM((_CHUNK, _D), jnp.float32)] * _NB
                   + [pltpu.SemaphoreType.DMA, pltpu.SemaphoreType.DMA]),
    compiler_params=pltpu.CompilerParams(use_tc_tiling_on_sc=False),
)

_scatter = pl.kernel(
    _scatter_body, mesh=_sc_mesh,
    out_type=jax.ShapeDtypeStruct((2, _NPAD, _D), jnp.float32),
    scratch_types=([pltpu.VMEM((_NCHUNK, _CHUNK), jnp.int32)]
                   + [pltpu.VMEM((_CHUNK, _D), jnp.float32)] * _NB
                   + [pltpu.SemaphoreType.DMA,
                      pltpu.VMEM_SHARED((_NPAD, _D), jnp.float32)]),
    compiler_params=pltpu.CompilerParams(use_tc_tiling_on_sc=False),
)


def _update_body(agg0_ref, agg1_ref, deg0_ref, deg1_ref, h_ref, root_ref,
                 bias_ref, wiht_ref, bih_ref, whht_ref, bhh_ref, h2_ref):
    h = h_ref[...]
    deg = jnp.maximum(deg0_ref[...] + deg1_ref[...], 1.0)
    agg = (agg0_ref[...] + agg1_ref[...]) / deg
    m = jnp.maximum(
        agg + jnp.dot(h, root_ref[...], preferred_element_type=jnp.float32)
        + bias_ref[...], 0.0)
    gi = jnp.dot(m, wiht_ref[...], preferred_element_type=jnp.float32) + bih_ref[...]
    gh = jnp.dot(h, whht_ref[...], preferred_element_type=jnp.float32) + bhh_ref[...]
    r = jax.nn.sigmoid(gi[:, :_D] + gh[:, :_D])
    z = jax.nn.sigmoid(gi[:, _D:2 * _D] + gh[:, _D:2 * _D])
    n = jnp.tanh(gi[:, 2 * _D:] + r * gh[:, 2 * _D:])
    h2_ref[...] = (1.0 - z) * n + z * h


def _epilogue_body(h_ref, w1t_ref, b1_ref, w2_ref, b2_ref, y_ref):
    t = jnp.maximum(
        jnp.dot(h_ref[...], w1t_ref[...], preferred_element_type=jnp.float32)
        + b1_ref[...], 0.0)
    y_ref[...] = jnp.sum(t * w2_ref[...], axis=1, keepdims=True) + b2_ref[...]


def _full(shape):
    return pl.BlockSpec(shape, lambda i: (0,) * len(shape))


def kernel(x, edge_index, edge_attr, W0, b0, Wn1, bn1, Wn2, bn2, root, bias,
           Wih, Whh, bih, bhh, W1, b1, W2, b2):
    pad = _EPAD - _E
    src = jnp.concatenate([edge_index[0], jnp.zeros((pad,), jnp.int32)])
    src3 = src.reshape(_NW, _NCHUNK, _CHUNK)
    dst = jnp.concatenate([edge_index[1],
                           jnp.full((pad,), _N, jnp.int32)])
    dst3 = dst.reshape(_NW, _NCHUNK, _CHUNK)
    ea_pad = jnp.concatenate(
        [edge_attr, jnp.zeros((pad, edge_attr.shape[1]), jnp.float32)])
    zeros_acc = jnp.zeros((_NPAD, _D), jnp.float32)
    ones_e = jnp.ones((_EPAD, _D), jnp.float32)

    out = pl.pallas_call(
        _prologue_body,
        grid=(_N // _BN,),
        in_specs=[pl.BlockSpec((_BN, 128), lambda i: (i, 0)),
                  _full((128, _D)), _full((1, _D))],
        out_specs=pl.BlockSpec((_BN, _D), lambda i: (i, 0)),
        out_shape=jax.ShapeDtypeStruct((_N, _D), jnp.float32),
    )(x, W0.T, b0.reshape(1, _D))

    hidden = pl.pallas_call(
        _hidden_body,
        grid=(_EPAD // _BE,),
        in_specs=[pl.BlockSpec((_BE, 16), lambda i: (i, 0)),
                  _full((16, _D)), _full((1, _D))],
        out_specs=pl.BlockSpec((_BE, _D), lambda i: (i, 0)),
        out_shape=jax.ShapeDtypeStruct((_EPAD, _D), jnp.float32),
    )(ea_pad, Wn1.T, bn1.reshape(1, _D))

    degp = _scatter(ones_e, dst3, zeros_acc)
    deg0 = degp[0, :_N]
    deg1 = degp[1, :_N]

    wn2t = Wn2.T
    bn2m = bn2.reshape(_D, _D)

    msg_call = pl.pallas_call(
        _msg_body,
        grid=(_EPAD // _BE,),
        in_specs=[pl.BlockSpec((_BE, _D), lambda i: (i, 0)),
                  pl.BlockSpec((_BE, _D), lambda i: (i, 0)),
                  _full((_D, _D * _D)), _full((_D, _D))],
        out_specs=pl.BlockSpec((_BE, _D), lambda i: (i, 0)),
        out_shape=jax.ShapeDtypeStruct((_EPAD, _D), jnp.float32),
    )

    upd_call = pl.pallas_call(
        _update_body,
        grid=(_N // _BN,),
        in_specs=[pl.BlockSpec((_BN, _D), lambda i: (i, 0)),
                  pl.BlockSpec((_BN, _D), lambda i: (i, 0)),
                  pl.BlockSpec((_BN, _D), lambda i: (i, 0)),
                  pl.BlockSpec((_BN, _D), lambda i: (i, 0)),
                  pl.BlockSpec((_BN, _D), lambda i: (i, 0)),
                  _full((_D, _D)), _full((1, _D)),
                  _full((_D, 3 * _D)), _full((1, 3 * _D)),
                  _full((_D, 3 * _D)), _full((1, 3 * _D))],
        out_specs=pl.BlockSpec((_BN, _D), lambda i: (i, 0)),
        out_shape=jax.ShapeDtypeStruct((_N, _D), jnp.float32),
    )

    rootm = root
    biasr = bias.reshape(1, _D)
    wiht = Wih.T
    bihr = bih.reshape(1, 3 * _D)
    whht = Whh.T
    bhhr = bhh.reshape(1, 3 * _D)

    for _ in range(3):
        out_src = _gather(out, src3)
        msg = msg_call(out_src, hidden, wn2t, bn2m)
        aggp = _scatter(msg, dst3, zeros_acc)
        out = upd_call(aggp[0, :_N], aggp[1, :_N], deg0, deg1, out,
                       rootm, biasr, wiht, bihr, whht, bhhr)

    y = pl.pallas_call(
        _epilogue_body,
        grid=(_N // _BN,),
        in_specs=[pl.BlockSpec((_BN, _D), lambda i: (i, 0)),
                  _full((_D, _D)), _full((1, _D)),
                  _full((1, _D)), _full((1, 1))],
        out_specs=pl.BlockSpec((_BN, 1), lambda i: (i, 0)),
        out_shape=jax.ShapeDtypeStruct((_N, 1), jnp.float32),
    )(out, W1.T, b1.reshape(1, _D), W2, b2.reshape(1, 1))
    return y


# speedup vs baseline: 4.1197x; 2.9629x over previous
import functools

import jax
import jax.numpy as jnp
from jax import lax
from jax.experimental import pallas as pl
from jax.experimental.pallas import tpu as pltpu
from jax.experimental.pallas import tpu_sc as plsc

_N = 10000
_E = 160000
_D = 32

_BN = 2000
_BE = 2048

_NW = 32
_CHUNK = 128
_NCHUNK = 40
_EPW = _CHUNK * _NCHUNK
_EPAD = _NW * _EPW
_NPAD = 10240
_STRIPE = _NPAD // 16


def _prologue_body(x_ref, w0t_ref, b0_ref, o_ref):
    o_ref[...] = jnp.maximum(
        jnp.dot(x_ref[...], w0t_ref[...], preferred_element_type=jnp.float32)
        + b0_ref[...], 0.0)


def _hidden_body(ea_ref, wn1t_ref, bn1_ref, o_ref):
    o_ref[...] = jnp.maximum(
        jnp.dot(ea_ref[...], wn1t_ref[...], preferred_element_type=jnp.float32)
        + bn1_ref[...], 0.0)


def _msg_body(osrc_ref, hid_ref, wflat_ref, tsel_ref, bn2m_ref, msg_ref):
    osrc = osrc_ref[...]
    hid = hid_ref[...]
    a = jnp.dot(osrc, wflat_ref[...], preferred_element_type=jnp.float32)
    b = jnp.dot(hid, tsel_ref[...], preferred_element_type=jnp.float32)
    u = a * b
    w = _D * _D
    while w > _D:
        w //= 2
        u = u[:, :w] + u[:, w:]
    msg_ref[...] = u + jnp.dot(osrc, bn2m_ref[...],
                               preferred_element_type=jnp.float32)


_NB = 8
_NROUND = _NCHUNK // _NB


def _gather_body(tab_hbm, idx_hbm, out_hbm, idx_v, *rest):
    rows = rest[:_NB]
    semg, sems = rest[_NB], rest[_NB + 1]
    wid = lax.axis_index("s") * 2 + lax.axis_index("c")
    base = wid * _EPW
    pltpu.sync_copy(idx_hbm.at[wid], idx_v)

    def round_body(r, carry):
        cps = [pltpu.async_copy(tab_hbm.at[idx_v.at[r * _NB + b]],
                                rows[b], semg) for b in range(_NB)]
        for c in cps:
            c.wait()
        outs = [pltpu.async_copy(
            rows[b],
            out_hbm.at[pl.ds(base + (r * _NB + b) * _CHUNK, _CHUNK)],
            sems) for b in range(_NB)]
        for c in outs:
            c.wait()
        return carry

    lax.fori_loop(0, _NROUND, round_body, 0)


def _scatter_body(msg_hbm, idx3_hbm, zeros_hbm, out_hbm, idx_v, *rest):
    rows = rest[:_NB]
    semg, acc = rest[_NB], rest[_NB + 1]
    cid = lax.axis_index("c")
    sid = lax.axis_index("s")
    wid = sid * 2 + cid
    pltpu.sync_copy(zeros_hbm.at[pl.ds(sid * _STRIPE, _STRIPE)],
                    acc.at[pl.ds(sid * _STRIPE, _STRIPE)])
    plsc.subcore_barrier()
    pltpu.sync_copy(idx3_hbm.at[wid], idx_v)

    def round_body(r, carry):
        cps = [pltpu.async_copy(
            msg_hbm.at[pl.ds(wid * _EPW + (r * _NB + b) * _CHUNK, _CHUNK)],
            rows[b], semg) for b in range(_NB)]
        for c in cps:
            c.wait()
        for b in range(_NB):
            pltpu.sync_copy(rows[b], acc.at[idx_v.at[r * _NB + b]], add=True)
        return carry

    lax.fori_loop(0, _NROUND, round_body, 0)
    plsc.subcore_barrier()
    pltpu.sync_copy(acc.at[pl.ds(sid * _STRIPE, _STRIPE)],
                    out_hbm.at[cid, pl.ds(sid * _STRIPE, _STRIPE)])


_sc_mesh = plsc.VectorSubcoreMesh(core_axis_name="c", subcore_axis_name="s")

_gather = pl.kernel(
    _gather_body, mesh=_sc_mesh,
    out_type=jax.ShapeDtypeStruct((_EPAD, _D), jnp.float32),
    scratch_types=([pltpu.VMEM((_NCHUNK, _CHUNK), jnp.int32)]
                   + [pltpu.VMEM((_CHUNK, _D), jnp.float32)] * _NB
                   + [pltpu.SemaphoreType.DMA, pltpu.SemaphoreType.DMA]),
    compiler_params=pltpu.CompilerParams(use_tc_tiling_on_sc=False),
)

_scatter = pl.kernel(
    _scatter_body, mesh=_sc_mesh,
    out_type=jax.ShapeDtypeStruct((2, _NPAD, _D), jnp.float32),
    scratch_types=([pltpu.VMEM((_NCHUNK, _CHUNK), jnp.int32)]
                   + [pltpu.VMEM((_CHUNK, _D), jnp.float32)] * _NB
                   + [pltpu.SemaphoreType.DMA,
                      pltpu.VMEM_SHARED((_NPAD, _D), jnp.float32)]),
    compiler_params=pltpu.CompilerParams(use_tc_tiling_on_sc=False),
)


def _update_body(agg0_ref, agg1_ref, deg0_ref, deg1_ref, h_ref, root_ref,
                 bias_ref, wiht_ref, bih_ref, whht_ref, bhh_ref, h2_ref):
    h = h_ref[...]
    deg = jnp.maximum(deg0_ref[...] + deg1_ref[...], 1.0)
    agg = (agg0_ref[...] + agg1_ref[...]) / deg
    m = jnp.maximum(
        agg + jnp.dot(h, root_ref[...], preferred_element_type=jnp.float32)
        + bias_ref[...], 0.0)
    gi = jnp.dot(m, wiht_ref[...], preferred_element_type=jnp.float32) + bih_ref[...]
    gh = jnp.dot(h, whht_ref[...], preferred_element_type=jnp.float32) + bhh_ref[...]
    r = jax.nn.sigmoid(gi[:, :_D] + gh[:, :_D])
    z = jax.nn.sigmoid(gi[:, _D:2 * _D] + gh[:, _D:2 * _D])
    n = jnp.tanh(gi[:, 2 * _D:] + r * gh[:, 2 * _D:])
    h2_ref[...] = (1.0 - z) * n + z * h


def _epilogue_body(h_ref, w1t_ref, b1_ref, w2_ref, b2_ref, y_ref):
    t = jnp.maximum(
        jnp.dot(h_ref[...], w1t_ref[...], preferred_element_type=jnp.float32)
        + b1_ref[...], 0.0)
    y_ref[...] = jnp.sum(t * w2_ref[...], axis=1, keepdims=True) + b2_ref[...]


def _full(shape):
    return pl.BlockSpec(shape, lambda i: (0,) * len(shape))


def kernel(x, edge_index, edge_attr, W0, b0, Wn1, bn1, Wn2, bn2, root, bias,
           Wih, Whh, bih, bhh, W1, b1, W2, b2):
    pad = _EPAD - _E
    src = jnp.concatenate([edge_index[0], jnp.zeros((pad,), jnp.int32)])
    src3 = src.reshape(_NW, _NCHUNK, _CHUNK)
    dst = jnp.concatenate([edge_index[1],
                           jnp.full((pad,), _N, jnp.int32)])
    dst3 = dst.reshape(_NW, _NCHUNK, _CHUNK)
    ea_pad = jnp.concatenate(
        [edge_attr, jnp.zeros((pad, edge_attr.shape[1]), jnp.float32)])
    zeros_acc = jnp.zeros((_NPAD, _D), jnp.float32)
    ones_e = jnp.ones((_EPAD, _D), jnp.float32)

    out = pl.pallas_call(
        _prologue_body,
        grid=(_N // _BN,),
        in_specs=[pl.BlockSpec((_BN, 128), lambda i: (i, 0)),
                  _full((128, _D)), _full((1, _D))],
        out_specs=pl.BlockSpec((_BN, _D), lambda i: (i, 0)),
        out_shape=jax.ShapeDtypeStruct((_N, _D), jnp.float32),
    )(x, W0.T, b0.reshape(1, _D))

    hidden = pl.pallas_call(
        _hidden_body,
        grid=(_EPAD // _BE,),
        in_specs=[pl.BlockSpec((_BE, 16), lambda i: (i, 0)),
                  _full((16, _D)), _full((1, _D))],
        out_specs=pl.BlockSpec((_BE, _D), lambda i: (i, 0)),
        out_shape=jax.ShapeDtypeStruct((_EPAD, _D), jnp.float32),
    )(ea_pad, Wn1.T, bn1.reshape(1, _D))

    degp = _scatter(ones_e, dst3, zeros_acc)
    deg0 = degp[0, :_N]
    deg1 = degp[1, :_N]

    wflat = Wn2.reshape(_D, _D, _D).transpose(0, 2, 1).reshape(_D, _D * _D)
    tsel = jnp.repeat(jnp.eye(_D, dtype=jnp.float32), _D, axis=1)
    bn2m = bn2.reshape(_D, _D)

    msg_call = pl.pallas_call(
        _msg_body,
        grid=(_EPAD // _BE,),
        in_specs=[pl.BlockSpec((_BE, _D), lambda i: (i, 0)),
                  pl.BlockSpec((_BE, _D), lambda i: (i, 0)),
                  _full((_D, _D * _D)), _full((_D, _D * _D)),
                  _full((_D, _D))],
        out_specs=pl.BlockSpec((_BE, _D), lambda i: (i, 0)),
        out_shape=jax.ShapeDtypeStruct((_EPAD, _D), jnp.float32),
    )

    upd_call = pl.pallas_call(
        _update_body,
        grid=(_N // _BN,),
        in_specs=[pl.BlockSpec((_BN, _D), lambda i: (i, 0)),
                  pl.BlockSpec((_BN, _D), lambda i: (i, 0)),
                  pl.BlockSpec((_BN, _D), lambda i: (i, 0)),
                  pl.BlockSpec((_BN, _D), lambda i: (i, 0)),
                  pl.BlockSpec((_BN, _D), lambda i: (i, 0)),
                  _full((_D, _D)), _full((1, _D)),
                  _full((_D, 3 * _D)), _full((1, 3 * _D)),
                  _full((_D, 3 * _D)), _full((1, 3 * _D))],
        out_specs=pl.BlockSpec((_BN, _D), lambda i: (i, 0)),
        out_shape=jax.ShapeDtypeStruct((_N, _D), jnp.float32),
    )

    rootm = root
    biasr = bias.reshape(1, _D)
    wiht = Wih.T
    bihr = bih.reshape(1, 3 * _D)
    whht = Whh.T
    bhhr = bhh.reshape(1, 3 * _D)

    for _ in range(3):
        out_src = _gather(out, src3)
        msg = msg_call(out_src, hidden, wflat, tsel, bn2m)
        aggp = _scatter(msg, dst3, zeros_acc)
        out = upd_call(aggp[0, :_N], aggp[1, :_N], deg0, deg1, out,
                       rootm, biasr, wiht, bihr, whht, bhhr)

    y = pl.pallas_call(
        _epilogue_body,
        grid=(_N // _BN,),
        in_specs=[pl.BlockSpec((_BN, _D), lambda i: (i, 0)),
                  _full((_D, _D)), _full((1, _D)),
                  _full((1, _D)), _full((1, 1))],
        out_specs=pl.BlockSpec((_BN, 1), lambda i: (i, 0)),
        out_shape=jax.ShapeDtypeStruct((_N, 1), jnp.float32),
    )(out, W1.T, b1.reshape(1, _D), W2, b2.reshape(1, 1))
    return y
